# baseline (speedup 1.0000x reference)
import jax
import jax.numpy as jnp
from jax import lax
from jax.experimental import pallas as pl
from jax.experimental.pallas import tpu as pltpu
from jax.experimental.pallas import tpu_sc as plsc

NC = 2
NS = 16
NW = NC * NS

E = 320000
ETC = 25600
ESC = E - ETC
D = 128
EPW = ESC // NW
CH = 80
NCH = EPW // CH
GPC = CH // 16
NBUF = 8


def _body(z_hbm, src_hbm, dst_hbm, out_hbm, idx_u, idx_v, rows_u, rows_v,
          out_v, sem):
    cid = lax.axis_index("c")
    sid = lax.axis_index("s")
    wid = sid * NC + cid
    e0 = wid * EPW

    pltpu.sync_copy(src_hbm.at[pl.ds(e0, EPW)], idx_u)
    pltpu.sync_copy(dst_hbm.at[pl.ds(e0, EPW)], idx_v)

    def fire(c, b):
        pltpu.async_copy(z_hbm.at[idx_u.at[pl.ds(c * CH, CH)]],
                         rows_u.at[b], sem.at[b])
        pltpu.async_copy(z_hbm.at[idx_v.at[pl.ds(c * CH, CH)]],
                         rows_v.at[b], sem.at[b])

    def drain(c, b):
        pltpu.make_async_copy(z_hbm.at[idx_u.at[pl.ds(c * CH, CH)]],
                              rows_u.at[b], sem.at[b]).wait()
        pltpu.make_async_copy(z_hbm.at[idx_v.at[pl.ds(c * CH, CH)]],
                              rows_v.at[b], sem.at[b]).wait()

    for i in range(NBUF - 1):
        fire(i, i)

    lane = lax.iota(jnp.int32, 16)
    rots = [(lane + s) % 16 for s in (1, 2, 4, 8)]

    def rot(x, perm):
        return lax.gather(
            x, perm[:, None],
            lax.GatherDimensionNumbers(
                offset_dims=(), collapsed_slice_dims=(0,),
                start_index_map=(0,)),
            (1,), mode=lax.GatherScatterMode.PROMISE_IN_BOUNDS)

    def chunk_body(c, carry):
        b = c % NBUF
        nc = c + NBUF - 1

        @pl.when(nc < NCH)
        def _():
            fire(nc, nc % NBUF)

        drain(c, b)

        def group_body(g, carry2):
            def edge_body(j, acc):
                e = g * 16 + j
                p = None
                for k in range(D // 32):
                    u2 = plsc.bitcast(rows_u[b, e, pl.ds(k * 16, 16)],
                                      jnp.bfloat16)
                    v2 = plsc.bitcast(rows_v[b, e, pl.ds(k * 16, 16)],
                                      jnp.bfloat16)
                    ta, tb = plsc.unpack(
                        u2 * v2, format=plsc.PackFormat.INTERLEAVED)
                    t = ta + tb
                    p = t if p is None else p + t
                for perm in rots:
                    p = p + rot(p, perm)
                return jnp.where(lane == j, p, acc)

            acc = lax.fori_loop(0, 16, edge_body,
                                jnp.zeros((16,), jnp.float32), unroll=4)
            out_v[pl.ds(c * CH + g * 16, 16)] = 1.0 / (1.0 + jnp.exp(-acc))
            return carry2

        lax.fori_loop(0, GPC, group_body, 0)
        return carry

    lax.fori_loop(0, NCH, chunk_body, 0)
    pltpu.sync_copy(out_v, out_hbm.at[pl.ds(e0, EPW)])


@jax.jit
def kernel(z, edge_index):
    ei = edge_index.astype(jnp.int32)
    zb = lax.bitcast_convert_type(
        z.astype(jnp.bfloat16).reshape(z.shape[0], D // 2, 2), jnp.int32)
    mesh = plsc.VectorSubcoreMesh(core_axis_name="c", subcore_axis_name="s")
    out_sc = pl.kernel(
        _body,
        out_type=jax.ShapeDtypeStruct((ESC,), jnp.float32),
        mesh=mesh,
        compiler_params=pltpu.CompilerParams(needs_layout_passes=False,
                                             use_tc_tiling_on_sc=False),
        scratch_types=[
            pltpu.VMEM((EPW,), jnp.int32),
            pltpu.VMEM((EPW,), jnp.int32),
            pltpu.VMEM((NBUF, CH, D // 2), jnp.int32),
            pltpu.VMEM((NBUF, CH, D // 2), jnp.int32),
            pltpu.VMEM((EPW,), jnp.float32),
            pltpu.SemaphoreType.DMA((NBUF,)),
        ],
    )(zb, ei[0, :ESC], ei[1, :ESC])
    u = jnp.take(z, ei[0, ESC:], axis=0)
    v = jnp.take(z, ei[1, ESC:], axis=0)
    out_tc = jax.nn.sigmoid(jnp.sum(u * v, axis=-1))
    return jnp.concatenate([out_sc, out_tc])


# speedup vs baseline: 1.6347x; 1.6347x over previous
import jax
import jax.numpy as jnp
from jax import lax
from jax.experimental import pallas as pl
from jax.experimental.pallas import tpu as pltpu
from jax.experimental.pallas import tpu_sc as plsc

NC = 2
NS = 16
NW = NC * NS

E = 320000
D = 128
EPW = E // NW
CH = 80
NCH = EPW // CH
GPC = CH // 16
NBUF = 4


def _body(z_hbm, src_hbm, dst_hbm, out_hbm, idx_u, idx_v, rows_u, rows_v,
          out_v, sem):
    cid = lax.axis_index("c")
    sid = lax.axis_index("s")
    wid = sid * NC + cid
    e0 = wid * EPW

    pltpu.sync_copy(src_hbm.at[pl.ds(e0, EPW)], idx_u)
    pltpu.sync_copy(dst_hbm.at[pl.ds(e0, EPW)], idx_v)

    def fire(c, b):
        pltpu.async_copy(z_hbm.at[idx_u.at[pl.ds(c * CH, CH)]],
                         rows_u.at[b], sem.at[b])
        pltpu.async_copy(z_hbm.at[idx_v.at[pl.ds(c * CH, CH)]],
                         rows_v.at[b], sem.at[b])

    def drain(c, b):
        pltpu.make_async_copy(z_hbm.at[idx_u.at[pl.ds(c * CH, CH)]],
                              rows_u.at[b], sem.at[b]).wait()
        pltpu.make_async_copy(z_hbm.at[idx_v.at[pl.ds(c * CH, CH)]],
                              rows_v.at[b], sem.at[b]).wait()

    for i in range(NBUF - 1):
        fire(i, i)

    lane = lax.iota(jnp.int32, 16)
    rots = [(lane + s) % 16 for s in (1, 2, 4, 8)]

    def rot(x, perm):
        return lax.gather(
            x, perm[:, None],
            lax.GatherDimensionNumbers(
                offset_dims=(), collapsed_slice_dims=(0,),
                start_index_map=(0,)),
            (1,), mode=lax.GatherScatterMode.PROMISE_IN_BOUNDS)

    def chunk_body(c, carry):
        b = c % NBUF
        nc = c + NBUF - 1

        @pl.when(nc < NCH)
        def _():
            fire(nc, nc % NBUF)

        drain(c, b)

        def group_body(g, carry2):
            def edge_body(j, acc):
                e = g * 16 + j
                p = None
                for k in range(D // 16):
                    u = rows_u[b, e, pl.ds(k * 16, 16)]
                    v = rows_v[b, e, pl.ds(k * 16, 16)]
                    t = u * v
                    p = t if p is None else p + t
                for perm in rots:
                    p = p + rot(p, perm)
                return jnp.where(lane == j, p, acc)

            acc = lax.fori_loop(0, 16, edge_body,
                                jnp.zeros((16,), jnp.float32), unroll=4)
            out_v[pl.ds(c * CH + g * 16, 16)] = 1.0 / (1.0 + jnp.exp(-acc))
            return carry2

        lax.fori_loop(0, GPC, group_body, 0)
        return carry

    lax.fori_loop(0, NCH, chunk_body, 0)
    pltpu.sync_copy(out_v, out_hbm.at[pl.ds(e0, EPW)])


@jax.jit
def kernel(z, edge_index):
    ei = edge_index.astype(jnp.int32)
    mesh = plsc.VectorSubcoreMesh(core_axis_name="c", subcore_axis_name="s")
    return pl.kernel(
        _body,
        out_type=jax.ShapeDtypeStruct((E,), jnp.float32),
        mesh=mesh,
        compiler_params=pltpu.CompilerParams(needs_layout_passes=False),
        scratch_types=[
            pltpu.VMEM((EPW,), jnp.int32),
            pltpu.VMEM((EPW,), jnp.int32),
            pltpu.VMEM((NBUF, CH, D), jnp.float32),
            pltpu.VMEM((NBUF, CH, D), jnp.float32),
            pltpu.VMEM((EPW,), jnp.float32),
            pltpu.SemaphoreType.DMA((NBUF,)),
        ],
    )(z, ei[0], ei[1])
